# baseline (speedup 1.0000x reference)
import jax
import jax.numpy as jnp
from jax import lax
from jax.experimental import pallas as pl
from jax.experimental.pallas import tpu as pltpu
from jax.experimental.pallas import tpu_sc as plsc

VOCAB_SIZE = 1000000
EMBED_DIM = 64
BATCH = 16384

NUM_CORES = 2
NUM_SUBCORES = 16
NUM_WORKERS = NUM_CORES * NUM_SUBCORES
TOTAL = 2 * BATCH
B_PER_W = TOTAL // NUM_WORKERS
CHUNKS = B_PER_W // 128
SLAB = 128
WINW = 4 * SLAB
FULL_LANES = (VOCAB_SIZE // SLAB) * SLAB
MAX_WSTART = FULL_LANES - 2 * WINW
LAST_SLAB = FULL_LANES


def _gather_body(sidx_hbm, ord_hbm, tableT_hbm, out_hbm,
                 sv, ov, win, tail, st, sem_g, sem_s):
  wid = lax.axis_index("s") * NUM_CORES + lax.axis_index("c")
  base = wid * B_PER_W
  pltpu.sync_copy(sidx_hbm.at[pl.ds(base, B_PER_W)], sv)
  pltpu.sync_copy(ord_hbm.at[wid], ov)
  pltpu.sync_copy(tableT_hbm.at[:, pl.ds(LAST_SLAB, 64)], tail)
  lanes16 = lax.iota(jnp.int32, 16)

  def extract_from(buf, width, row, colbase):
    cols = jnp.broadcast_to(
        jnp.minimum(colbase, width - 1), (16,)).astype(jnp.int32)
    for q in range(4):
      vals = plsc.load_gather(buf, [lanes16 + (16 * q), cols])
      st[row, pl.ds(16 * q, 16)] = vals

  def extract(row, colbase):
    extract_from(win, 2 * WINW, row, colbase)

  def do_chunk(c, carry):
    def do_group(g, carry2):
      vec = sv[pl.ds(c * 128 + g * 16, 16)]
      wstart = jnp.minimum((vec[0] // SLAB) * SLAB, MAX_WSTART)
      reach = vec[15] - wstart
      near = reach < WINW
      mid = jnp.logical_and(reach >= WINW, reach < 2 * WINW)
      far = reach >= 2 * WINW

      @pl.when(near)
      def _():
        pltpu.async_copy(
            tableT_hbm.at[:, pl.ds(wstart, WINW)],
            win.at[:, pl.ds(0, WINW)], sem_g).wait()

      @pl.when(mid)
      def _():
        h1 = pltpu.async_copy(
            tableT_hbm.at[:, pl.ds(wstart, WINW)],
            win.at[:, pl.ds(0, WINW)], sem_g)
        h2 = pltpu.async_copy(
            tableT_hbm.at[:, pl.ds(wstart + WINW, WINW)],
            win.at[:, pl.ds(WINW, WINW)], sem_g)
        h1.wait()
        h2.wait()

      for j in range(16):
        extract(g * 16 + j, vec[j] - wstart)

      @pl.when(far)
      def _():
        for h in range(2):
          for j in range(8):
            vj = vec[8 * h + j]
            sstart = (vj // SLAB) * SLAB

            @pl.when(sstart < LAST_SLAB)
            def _():
              pltpu.async_copy(
                  tableT_hbm.at[:, pl.ds(sstart, SLAB)],
                  win.at[:, pl.ds(j * SLAB, SLAB)], sem_g).wait()

          for j in range(8):
            vj = vec[8 * h + j]
            sstart = (vj // SLAB) * SLAB
            row = g * 16 + 8 * h + j

            @pl.when(vj < LAST_SLAB)
            def _():
              extract(row, j * SLAB + vj - sstart)

            @pl.when(vj >= LAST_SLAB)
            def _():
              extract_from(tail, 64, row, vj - LAST_SLAB)

      return carry2

    lax.fori_loop(0, 8, do_group, 0)
    pltpu.async_copy(st, out_hbm.at[ov.at[c]], sem_s).wait()
    return carry

  lax.fori_loop(0, CHUNKS, do_chunk, 0)


@jax.jit
def kernel(target, other, embed_table):
  mesh = plsc.VectorSubcoreMesh(
      core_axis_name="c", subcore_axis_name="s",
      num_cores=NUM_CORES, num_subcores=NUM_SUBCORES)
  run = pl.kernel(
      _gather_body,
      out_type=jax.ShapeDtypeStruct((TOTAL, 128), jnp.float32),
      mesh=mesh,
      scratch_types=[
          pltpu.VMEM((B_PER_W,), jnp.int32),
          pltpu.VMEM((CHUNKS, 128), jnp.int32),
          pltpu.VMEM((EMBED_DIM, 2 * WINW), jnp.float32),
          pltpu.VMEM((EMBED_DIM, 64), jnp.float32),
          pltpu.VMEM((128, 128), jnp.float32),
          pltpu.SemaphoreType.DMA,
          pltpu.SemaphoreType.DMA,
      ],
      compiler_params=pltpu.CompilerParams(needs_layout_passes=False),
  )
  idx_all = jnp.concatenate(
      [target.astype(jnp.int32), other.astype(jnp.int32)])
  order = jnp.argsort(idx_all).astype(jnp.int32)
  sorted_idx = jnp.take(idx_all, order)
  order3 = order.reshape(NUM_WORKERS, CHUNKS, 128)
  out = run(sorted_idx, order3, embed_table.T)
  return (out[:BATCH, :EMBED_DIM], out[BATCH:, :EMBED_DIM])


# speedup vs baseline: 1.0593x; 1.0593x over previous
import jax
import jax.numpy as jnp
from jax import lax
from jax.experimental import pallas as pl
from jax.experimental.pallas import tpu as pltpu
from jax.experimental.pallas import tpu_sc as plsc

VOCAB_SIZE = 1000000
EMBED_DIM = 64
BATCH = 16384

NUM_CORES = 2
NUM_SUBCORES = 16
NUM_WORKERS = NUM_CORES * NUM_SUBCORES
TOTAL = 2 * BATCH
B_PER_W = TOTAL // NUM_WORKERS
CHUNKS = B_PER_W // 128
SLAB = 128
WINW = 4 * SLAB
FULL_LANES = (VOCAB_SIZE // SLAB) * SLAB
MAX_WSTART = FULL_LANES - 2 * WINW
LAST_SLAB = FULL_LANES


def _gather_body(sidx_hbm, ord_hbm, tableT_hbm, out_hbm,
                 sv, ov, win, tail, st, sem_g, sem_s):
  wid = lax.axis_index("s") * NUM_CORES + lax.axis_index("c")
  base = wid * B_PER_W
  pltpu.sync_copy(sidx_hbm.at[pl.ds(base, B_PER_W)], sv)
  pltpu.sync_copy(ord_hbm.at[wid], ov)
  pltpu.sync_copy(tableT_hbm.at[:, pl.ds(LAST_SLAB, 64)], tail)
  lanes16 = lax.iota(jnp.int32, 16)

  def extract_from(buf, width, row, colbase):
    cols = jnp.broadcast_to(
        jnp.minimum(colbase, width - 1), (16,)).astype(jnp.int32)
    for q in range(4):
      vals = plsc.load_gather(buf, [lanes16 + (16 * q), cols])
      st[row, pl.ds(16 * q, 16)] = vals

  def extract(row, colbase):
    extract_from(win, 2 * WINW, row, colbase)

  def do_chunk(c, lw_in):
    def do_group(g, lw):
      vec = sv[pl.ds(c * 128 + g * 16, 16)]
      need = jnp.logical_or(vec[0] < lw, vec[15] >= lw + 2 * WINW)
      wstart = pl.multiple_of(
          jnp.where(
              need, jnp.minimum((vec[0] // SLAB) * SLAB, MAX_WSTART), lw),
          SLAB)

      @pl.when(need)
      def _():
        h1 = pltpu.async_copy(
            tableT_hbm.at[:, pl.ds(wstart, WINW)],
            win.at[:, pl.ds(0, WINW)], sem_g)
        h2 = pltpu.async_copy(
            tableT_hbm.at[:, pl.ds(wstart + WINW, WINW)],
            win.at[:, pl.ds(WINW, WINW)], sem_g)
        h1.wait()
        h2.wait()

      far = vec[15] - wstart >= 2 * WINW

      for j in range(16):
        extract(g * 16 + j, vec[j] - wstart)

      @pl.when(far)
      def _():
        for h in range(2):
          for j in range(8):
            vj = vec[8 * h + j]
            sstart = (vj // SLAB) * SLAB

            @pl.when(sstart < LAST_SLAB)
            def _():
              pltpu.async_copy(
                  tableT_hbm.at[:, pl.ds(sstart, SLAB)],
                  win.at[:, pl.ds(j * SLAB, SLAB)], sem_g).wait()

          for j in range(8):
            vj = vec[8 * h + j]
            sstart = (vj // SLAB) * SLAB
            row = g * 16 + 8 * h + j

            @pl.when(vj < LAST_SLAB)
            def _():
              extract(row, j * SLAB + vj - sstart)

            @pl.when(vj >= LAST_SLAB)
            def _():
              extract_from(tail, 64, row, vj - LAST_SLAB)

      return jnp.where(far, jnp.int32(-16 * SLAB), wstart)

    lw_out = lax.fori_loop(0, 8, do_group, lw_in)
    pltpu.async_copy(st, out_hbm.at[ov.at[c]], sem_s).wait()
    return lw_out

  lax.fori_loop(0, CHUNKS, do_chunk, jnp.int32(-16 * SLAB))


@jax.jit
def kernel(target, other, embed_table):
  mesh = plsc.VectorSubcoreMesh(
      core_axis_name="c", subcore_axis_name="s",
      num_cores=NUM_CORES, num_subcores=NUM_SUBCORES)
  run = pl.kernel(
      _gather_body,
      out_type=jax.ShapeDtypeStruct((TOTAL, 128), jnp.float32),
      mesh=mesh,
      scratch_types=[
          pltpu.VMEM((B_PER_W,), jnp.int32),
          pltpu.VMEM((CHUNKS, 128), jnp.int32),
          pltpu.VMEM((EMBED_DIM, 2 * WINW), jnp.float32),
          pltpu.VMEM((EMBED_DIM, 64), jnp.float32),
          pltpu.VMEM((128, 128), jnp.float32),
          pltpu.SemaphoreType.DMA,
          pltpu.SemaphoreType.DMA,
      ],
      compiler_params=pltpu.CompilerParams(needs_layout_passes=False),
  )
  idx_all = jnp.concatenate(
      [target.astype(jnp.int32), other.astype(jnp.int32)])
  order = jnp.argsort(idx_all).astype(jnp.int32)
  sorted_idx = jnp.take(idx_all, order)
  order3 = order.reshape(NUM_WORKERS, CHUNKS, 128)
  out = run(sorted_idx, order3, embed_table.T)
  return (out[:BATCH, :EMBED_DIM], out[BATCH:, :EMBED_DIM])
